# Initial kernel scaffold; baseline (speedup 1.0000x reference)
import jax
import jax.numpy as jnp
from jax.experimental import pallas as pl


def kernel(x, table, fc_w, fc_b):
    raise NotImplementedError("write your pallas kernel here")


# speedup vs baseline: 2.6285x; 2.6285x over previous
import functools

import jax
import jax.numpy as jnp
from jax import lax
from jax.experimental import pallas as pl
from jax.experimental.pallas import tpu as pltpu
from jax.experimental.pallas import tpu_sc as plsc

B = 4096
L = 200
D = 32
NC = 2
NS = 16
NW = NC * NS
BPW = B // NW
LANES = 16


def _body(xt_hbm, table_hbm, wb_hbm, out_hbm,
          xt_v, acc_v, cnt_v, out_v, wb_v, t0_v, sem):
    wid = lax.axis_index("s") * NC + lax.axis_index("c")
    base = wid * BPW

    pltpu.sync_copy(xt_hbm.at[:, pl.ds(base, BPW)], xt_v)
    pltpu.sync_copy(wb_hbm, wb_v)
    pltpu.sync_copy(table_hbm.at[pl.ds(0, 1)], t0_v)

    zf = jnp.zeros((LANES,), jnp.float32)

    @pl.loop(0, BPW)
    def _zero(i):
        acc_v[i, pl.ds(0, LANES)] = zf
        acc_v[i, pl.ds(LANES, LANES)] = zf

    for c in range(BPW // LANES):
        cnt_v[c] = zf

    @pl.loop(0, L)
    def _fire(l):
        pltpu.async_copy(table_hbm.at[xt_v.at[l]], acc_v, sem, add=True)
        for c in range(BPW // LANES):
            v = xt_v[l, pl.ds(c * LANES, LANES)]
            cnt_v[c] = cnt_v[c] + jnp.where(v == 0, 1.0, 0.0).astype(jnp.float32)

    @pl.loop(0, L)
    def _drain(l):
        pltpu.make_async_copy(table_hbm.at[pl.ds(0, BPW)], acc_v, sem).wait()

    w0 = wb_v[pl.ds(0, LANES)]
    w1 = wb_v[pl.ds(LANES, LANES)]
    sv = t0_v[0, pl.ds(0, LANES)] * w0 + t0_v[0, pl.ds(LANES, LANES)] * w1
    score0 = sv[0]
    for i in range(1, LANES):
        score0 = score0 + sv[i]
    bias = wb_v[pl.ds(2 * LANES, LANES)][0]
    inv = jnp.float32(1.0 / L)
    iot = lax.iota(jnp.int32, LANES)
    for c in range(BPW // LANES):
        rows = iot + (c * LANES)
        o = zf
        for d in range(D):
            col = jnp.full((LANES,), d, jnp.int32)
            wd = w0[d] if d < LANES else w1[d - LANES]
            o = o + plsc.load_gather(acc_v, [rows, col]) * wd
        out_v[pl.ds(c * LANES, LANES)] = (o - cnt_v[c] * score0) * inv + bias

    pltpu.sync_copy(out_v, out_hbm.at[pl.ds(base, BPW)])


@jax.jit
def kernel(x, table, fc_w, fc_b):
    xt = x.T
    wb = jnp.concatenate(
        [fc_w.reshape(D), fc_b.reshape(1),
         jnp.zeros((LANES - 1,), jnp.float32)])
    mesh = plsc.VectorSubcoreMesh(
        core_axis_name="c", subcore_axis_name="s",
        num_cores=NC, num_subcores=NS)
    run = pl.kernel(
        _body,
        out_type=jax.ShapeDtypeStruct((B,), jnp.float32),
        mesh=mesh,
        compiler_params=pltpu.CompilerParams(
            needs_layout_passes=False, use_tc_tiling_on_sc=False),
        scratch_types=[
            pltpu.VMEM((L, BPW), jnp.int32),
            pltpu.VMEM((BPW, D), jnp.float32),
            pltpu.VMEM((BPW // LANES, LANES), jnp.float32),
            pltpu.VMEM((BPW,), jnp.float32),
            pltpu.VMEM((D + LANES,), jnp.float32),
            pltpu.VMEM((1, D), jnp.float32),
            pltpu.SemaphoreType.DMA,
        ],
    )
    return run(xt, table, wb)


# speedup vs baseline: 2.6318x; 1.0013x over previous
import functools

import jax
import jax.numpy as jnp
from jax import lax
from jax.experimental import pallas as pl
from jax.experimental.pallas import tpu as pltpu
from jax.experimental.pallas import tpu_sc as plsc

B = 4096
L = 200
D = 32
NC = 2
NS = 16
NW = NC * NS
BPW = B // NW
LANES = 16


def _body(x_hbm, table_hbm, wb_hbm, out_hbm,
          xb_v, xt_v, acc_v, cnt_v, out_v, wb_v, t0_v, sem):
    wid = lax.axis_index("s") * NC + lax.axis_index("c")
    base = wid * BPW

    pltpu.sync_copy(x_hbm.at[pl.ds(base, BPW), :], xb_v)
    pltpu.sync_copy(wb_hbm, wb_v)
    pltpu.sync_copy(table_hbm.at[pl.ds(0, 1)], t0_v)

    zf = jnp.zeros((LANES,), jnp.float32)

    @pl.loop(0, BPW)
    def _zero(i):
        acc_v[i, pl.ds(0, LANES)] = zf
        acc_v[i, pl.ds(LANES, LANES)] = zf

    for c in range(BPW // LANES):
        cnt_v[c] = zf

    iot = lax.iota(jnp.int32, LANES)
    one = jnp.ones((LANES,), jnp.float32)

    @pl.loop(0, L)
    def _fire(l):
        col = jnp.broadcast_to(l, (LANES,)).astype(jnp.int32)
        for c in range(BPW // LANES):
            v = plsc.load_gather(xb_v, [iot + (c * LANES), col])
            xt_v[l, pl.ds(c * LANES, LANES)] = v
            cnt_v[c] = cnt_v[c] + jnp.where(v == 0, one, 0.0)
        pltpu.async_copy(table_hbm.at[xt_v.at[l]], acc_v, sem, add=True)

    @pl.loop(0, L)
    def _drain(l):
        pltpu.make_async_copy(table_hbm.at[pl.ds(0, BPW)], acc_v, sem).wait()

    w0 = wb_v[pl.ds(0, LANES)]
    w1 = wb_v[pl.ds(LANES, LANES)]
    sv = t0_v[0, pl.ds(0, LANES)] * w0 + t0_v[0, pl.ds(LANES, LANES)] * w1
    score0 = sv[0]
    for i in range(1, LANES):
        score0 = score0 + sv[i]
    bias = wb_v[pl.ds(2 * LANES, LANES)][0]
    inv = jnp.float32(1.0 / L)
    iot = lax.iota(jnp.int32, LANES)
    for c in range(BPW // LANES):
        rows = iot + (c * LANES)
        o = zf
        for d in range(D):
            col = jnp.full((LANES,), d, jnp.int32)
            wd = w0[d] if d < LANES else w1[d - LANES]
            o = o + plsc.load_gather(acc_v, [rows, col]) * wd
        out_v[pl.ds(c * LANES, LANES)] = (o - cnt_v[c] * score0) * inv + bias

    pltpu.sync_copy(out_v, out_hbm.at[pl.ds(base, BPW)])


@jax.jit
def kernel(x, table, fc_w, fc_b):
    wb = jnp.concatenate(
        [fc_w.reshape(D), fc_b.reshape(1),
         jnp.zeros((LANES - 1,), jnp.float32)])
    mesh = plsc.VectorSubcoreMesh(
        core_axis_name="c", subcore_axis_name="s",
        num_cores=NC, num_subcores=NS)
    run = pl.kernel(
        _body,
        out_type=jax.ShapeDtypeStruct((B,), jnp.float32),
        mesh=mesh,
        compiler_params=pltpu.CompilerParams(
            needs_layout_passes=False, use_tc_tiling_on_sc=False),
        scratch_types=[
            pltpu.VMEM((BPW, L), jnp.int32),
            pltpu.VMEM((L, BPW), jnp.int32),
            pltpu.VMEM((BPW, D), jnp.float32),
            pltpu.VMEM((BPW // LANES, LANES), jnp.float32),
            pltpu.VMEM((BPW,), jnp.float32),
            pltpu.VMEM((D + LANES,), jnp.float32),
            pltpu.VMEM((1, D), jnp.float32),
            pltpu.SemaphoreType.DMA,
        ],
    )
    return run(x, table, wb)
